# baseline (speedup 1.0000x reference)
import math

import jax
import jax.numpy as jnp
from jax.experimental import pallas as pl
from jax.experimental.pallas import tpu as pltpu

_SQRT_HALF = 1.0 / math.sqrt(2.0)


def _mlp_kernel(x_ref, w1_ref, b1_ref, w2_ref, b2_ref, o_ref,
                w1b_ref, w2b_ref):
    @pl.when(pl.program_id(0) == 0)
    def _():
        w1b_ref[...] = w1_ref[...].astype(jnp.bfloat16)
        w2b_ref[...] = w2_ref[...].astype(jnp.bfloat16)

    nb, n, _ = x_ref.shape
    xcat = jnp.concatenate([x_ref[b].astype(jnp.bfloat16)
                            for b in range(nb)], axis=0)
    h = jnp.dot(xcat, w1b_ref[...], preferred_element_type=jnp.float32)
    h = h + b1_ref[...]
    h = 0.5 * h * (1.0 + jax.lax.erf(h * _SQRT_HALF))
    out = jnp.dot(h.astype(jnp.bfloat16), w2b_ref[...],
                  preferred_element_type=jnp.float32)
    out = out + b2_ref[...]
    for b in range(nb):
        o_ref[b] = out[b * n:(b + 1) * n].astype(o_ref.dtype)


def _mlp(x, w1, b1, w2, b2, *, tile_b=4):
    B, N, C_in = x.shape
    C_hid = w1.shape[1]
    C_out = w2.shape[1]

    b1_2d = b1.reshape(1, C_hid).astype(jnp.float32)
    b2_2d = b2.reshape(1, C_out).astype(jnp.float32)

    grid = (B // tile_b,)

    return pl.pallas_call(
        _mlp_kernel,
        out_shape=jax.ShapeDtypeStruct((B, N, C_out), x.dtype),
        grid=grid,
        in_specs=[
            pl.BlockSpec((tile_b, N, C_in), lambda i: (i, 0, 0)),
            pl.BlockSpec((C_in, C_hid), lambda i: (0, 0)),
            pl.BlockSpec((1, C_hid), lambda i: (0, 0)),
            pl.BlockSpec((C_hid, C_out), lambda i: (0, 0)),
            pl.BlockSpec((1, C_out), lambda i: (0, 0)),
        ],
        out_specs=pl.BlockSpec((tile_b, N, C_out), lambda i: (i, 0, 0)),
        scratch_shapes=[
            pltpu.VMEM((C_in, C_hid), jnp.bfloat16),
            pltpu.VMEM((C_hid, C_out), jnp.bfloat16),
        ],
        compiler_params=pltpu.CompilerParams(
            dimension_semantics=("arbitrary",),
            vmem_limit_bytes=56 * 1024 * 1024),
    )(x, w1, b1_2d, w2, b2_2d)


def kernel(x, w1, b1, w2, b2):
    return _mlp(x, w1, b1, w2, b2)
